# Initial kernel scaffold; baseline (speedup 1.0000x reference)
import jax
import jax.numpy as jnp
from jax.experimental import pallas as pl


def kernel(hidden_states, gate_w, expert_gate_w, expert_up_w, expert_down_w, shared_gate_w, shared_up_w, shared_down_w, shared_expert_gate_w):
    raise NotImplementedError("write your pallas kernel here")


# speedup vs baseline: 1.0207x; 1.0207x over previous
import jax
import jax.numpy as jnp
from jax.experimental import pallas as pl
from jax.experimental.pallas import tpu as pltpu

_E = 8
_D = 2048
_DFF = 1408
_SFF = 5632

_FCH = 352
_DCH = 256
_SCH = 256
_TCH = 512


def _router_body(xt_ref, gw_ref, ewt_ref):
    xt = xt_ref[...]
    gw = gw_ref[...]
    logits = jax.lax.dot_general(
        gw, xt, (((1,), (0,)), ((), ())), preferred_element_type=jnp.float32)
    iota = jax.lax.broadcasted_iota(jnp.int32, logits.shape, 0)
    m1 = jnp.max(logits, axis=0, keepdims=True)
    i1 = jnp.min(jnp.where(logits == m1, iota, _E), axis=0, keepdims=True)
    oh1 = iota == i1
    l2 = jnp.where(oh1, -jnp.inf, logits)
    m2 = jnp.max(l2, axis=0, keepdims=True)
    i2 = jnp.min(jnp.where(l2 == m2, iota, _E), axis=0, keepdims=True)
    oh2 = iota == i2
    w1 = 1.0 / (1.0 + jnp.exp(m2 - m1))
    w2 = 1.0 - w1
    ewt_ref[...] = jnp.where(oh1, w1, 0.0) + jnp.where(oh2, w2, 0.0)


def _moe_body(xt_ref, ewt_ref, ge_ref, ue_ref, de_ref, out_ref, ht_ref):
    e = pl.program_id(0)
    p = pl.program_id(1)
    xt = xt_ref[...]

    @pl.when(p < 4)
    def _():
        ge = ge_ref[0].astype(jnp.bfloat16)
        ue = ue_ref[0].astype(jnp.bfloat16)
        g = jax.lax.dot_general(
            ge, xt, (((1,), (0,)), ((), ())),
            preferred_element_type=jnp.float32)
        u = jax.lax.dot_general(
            ue, xt, (((1,), (0,)), ((), ())),
            preferred_element_type=jnp.float32)
        w = ewt_ref[0]
        ht = (g * jax.nn.sigmoid(g) * u * w).astype(jnp.bfloat16)
        row = pl.multiple_of(p * _FCH, 32)
        ht_ref[pl.ds(row, _FCH), :] = ht

    @pl.when(p >= 4)
    def _():
        de = de_ref[0].astype(jnp.bfloat16)
        tmp = jax.lax.dot_general(
            de, ht_ref[...], (((1,), (0,)), ((), ())),
            preferred_element_type=jnp.float32)
        row = pl.multiple_of((p - 4) * _DCH, _DCH)

        @pl.when(e == 0)
        def _():
            out_ref[pl.ds(row, _DCH), :] = tmp

        @pl.when(e > 0)
        def _():
            out_ref[pl.ds(row, _DCH), :] += tmp


def _shared_body(xb_ref, wg_ref, wu_ref, wd_ref, wsg_ref, out_ref, sig_ref):
    s = pl.program_id(0)
    xb = xb_ref[...]

    @pl.when(s == 0)
    def _():
        xf = xb.astype(jnp.float32)
        logit = jnp.sum(xf * wsg_ref[...], axis=1, keepdims=True)
        sig_ref[...] = jax.nn.sigmoid(logit)

    wg = wg_ref[...].astype(jnp.bfloat16)
    wu = wu_ref[...].astype(jnp.bfloat16)
    wd = wd_ref[...].astype(jnp.bfloat16)
    g = jax.lax.dot_general(
        xb, wg, (((1,), (1,)), ((), ())), preferred_element_type=jnp.float32)
    u = jax.lax.dot_general(
        xb, wu, (((1,), (1,)), ((), ())), preferred_element_type=jnp.float32)
    h = (g * jax.nn.sigmoid(g) * u * sig_ref[...]).astype(jnp.bfloat16)
    for k in range(4):
        hk = h[k * _TCH:(k + 1) * _TCH, :]
        tmp = jax.lax.dot_general(
            hk, wd, (((1,), (1,)), ((), ())),
            preferred_element_type=jnp.float32)

        @pl.when(s == 0)
        def _():
            out_ref[k * _TCH:(k + 1) * _TCH, :] = tmp

        @pl.when(s > 0)
        def _():
            out_ref[k * _TCH:(k + 1) * _TCH, :] += tmp


def kernel(hidden_states, gate_w, expert_gate_w, expert_up_w, expert_down_w,
           shared_gate_w, shared_up_w, shared_down_w, shared_expert_gate_w):
    b, seq, d = hidden_states.shape
    t = b * seq
    x = hidden_states.reshape(t, d)
    xt32 = jnp.swapaxes(x, 0, 1)
    xt = xt32.astype(jnp.bfloat16)
    xb = x.astype(jnp.bfloat16)

    ewt = pl.pallas_call(
        _router_body,
        out_shape=jax.ShapeDtypeStruct((_E, t), jnp.float32),
    )(xt32, gate_w)
    ewt3 = ewt.reshape(_E, 1, t)

    out_t = pl.pallas_call(
        _moe_body,
        grid=(_E, 12),
        in_specs=[
            pl.BlockSpec((_D, t), lambda e, p: (0, 0)),
            pl.BlockSpec((1, 1, t), lambda e, p: (e, 0, 0)),
            pl.BlockSpec((1, _FCH, _D),
                         lambda e, p: (e, jnp.minimum(p, 3), 0)),
            pl.BlockSpec((1, _FCH, _D),
                         lambda e, p: (e, jnp.minimum(p, 3), 0)),
            pl.BlockSpec((1, _DCH, _DFF),
                         lambda e, p: (e, jnp.maximum(p - 4, 0), 0)),
        ],
        out_specs=pl.BlockSpec((_D, t), lambda e, p: (0, 0)),
        out_shape=jax.ShapeDtypeStruct((_D, t), jnp.float32),
        scratch_shapes=[pltpu.VMEM((_DFF, t), jnp.bfloat16)],
        compiler_params=pltpu.CompilerParams(
            vmem_limit_bytes=64 * 1024 * 1024),
    )(xt, ewt3, expert_gate_w, expert_up_w, expert_down_w)

    n_s = _SFF // _SCH
    shared_out = pl.pallas_call(
        _shared_body,
        grid=(n_s,),
        in_specs=[
            pl.BlockSpec((t, _D), lambda s: (0, 0)),
            pl.BlockSpec((_SCH, _D), lambda s: (s, 0)),
            pl.BlockSpec((_SCH, _D), lambda s: (s, 0)),
            pl.BlockSpec((_D, _SCH), lambda s: (0, s)),
            pl.BlockSpec((1, _D), lambda s: (0, 0)),
        ],
        out_specs=pl.BlockSpec((t, _D), lambda s: (0, 0)),
        out_shape=jax.ShapeDtypeStruct((t, _D), jnp.float32),
        scratch_shapes=[pltpu.VMEM((t, 1), jnp.float32)],
        compiler_params=pltpu.CompilerParams(
            vmem_limit_bytes=64 * 1024 * 1024),
    )(xb, shared_gate_w, shared_up_w, shared_down_w, shared_expert_gate_w)

    return (jnp.swapaxes(out_t, 0, 1) + shared_out).reshape(b, seq, d)
